# baseline (speedup 1.0000x reference)
import functools

import jax
import jax.numpy as jnp
from jax import lax
from jax.experimental import pallas as pl
from jax.experimental.pallas import tpu as pltpu
from jax.experimental.pallas import tpu_sc as plsc

_N = 1_000_000
_K = 64
_B = 16384
_RATING_MEAN = 3.5

_NC = 2
_NS = 16
_L = 16
_NW = _NC * _NS
_BPW = _B // _NW

_PW = 384
_NLANE = 1_000_064
_NPAN = 2605
_LASTP = _NPAN - 1
_LASTLO = _NLANE - _PW
_PPW = _NPAN // _NW
_PEXTRA = _NPAN - _PPW * _NW
_DUMP = _B
_GROWS = _B + _L
_GK = 128
_FLUSH = 64
_RING = 3
_SEC = 2048


def _panel_lo(p):
    return jnp.where(p >= _LASTP, _LASTLO, p * _PW)


def _splat_lane(v, i):
    idx = jnp.broadcast_to(i.astype(jnp.int32), (_L,))[:, None]
    return lax.gather(
        v, idx,
        dimension_numbers=lax.GatherDimensionNumbers(
            offset_dims=(), collapsed_slice_dims=(0,), start_index_map=(0,)),
        slice_sizes=(1,), mode=lax.GatherScatterMode.PROMISE_IN_BOUNDS)


def _gather_body(user_h, item_h, qt_h, pt_h, qg_h, pg_h,
                 all_idx, listu, listb, panels, cols, bflat, b2d, tmpu, tmpb,
                 psems, ssems, asems):
    wid = lax.axis_index("s") * _NC + lax.axis_index("c")
    pstart = wid * _PPW + jnp.minimum(wid, _PEXTRA)
    pcnt = _PPW + jnp.where(wid < _PEXTRA, 1, 0)
    wlo = _panel_lo(pstart)
    last_p = pstart + pcnt - 1
    whi = jnp.where(last_p >= _LASTP, _NLANE, (last_p + 1) * _PW)
    lane = lax.iota(jnp.int32, _L)

    cnts = []
    for listref, idx_h in ((listu, user_h), (listb, item_h)):
        def fire_sec(s, slot, idx_h=idx_h):
            return pltpu.async_copy(
                idx_h.at[pl.ds(s * _SEC, _SEC)], all_idx.at[slot],
                asems.at[slot])

        cnt = jnp.int32(0)
        fire_sec(0, 0)
        for s in range(_B // _SEC):
            slot = s % 2
            if s + 1 < _B // _SEC:
                fire_sec(s + 1, (s + 1) % 2)
            pltpu.make_async_copy(
                idx_h.at[pl.ds(s * _SEC, _SEC)], all_idx.at[slot],
                asems.at[slot]).wait()

            def scan_body(c, cnt, s=s, slot=slot, listref=listref):
                off = pl.multiple_of(c * _L, _L)
                v = all_idx[slot, pl.ds(off, _L)] - 1
                v = jnp.where(v < 0, _N - 1, v)
                m = (v >= wlo) & (v < whi)
                packed = ((v - wlo) << 14) | (s * _SEC + off + lane)
                plsc.store_compressed(listref.at[pl.ds(cnt, _L)], packed,
                                      mask=m)
                return cnt + plsc.all_reduce_population_count(m)[0]

            cnt = lax.fori_loop(0, _SEC // _L, scan_body, cnt, unroll=2)
        listref[pl.ds(cnt, _L)] = jnp.full((_L,), jnp.int32(0x3FFFFFFF))
        listref[pl.ds(cnt + _L, _L)] = jnp.full((_L,), jnp.int32(0x3FFFFFFF))
        cnts.append(cnt)

    for tbl_h, out_h, listref, cnt in ((qt_h, qg_h, listu, cnts[0]),
                                       (pt_h, pg_h, listb, cnts[1])):
        npair = (cnt + 2 * _L - 1) // (2 * _L)

        def fire(p, slot):
            lo = pl.multiple_of(_panel_lo(p), 128)
            for q in range(4):
                pltpu.async_copy(
                    tbl_h.at[pl.ds(q * _K // 4, _K // 4), pl.ds(lo, _PW)],
                    panels.at[slot, pl.ds(q * _K // 4, _K // 4)],
                    psems.at[slot])

        def drain_flush(fs):
            pltpu.make_async_copy(
                cols.at[fs], out_h.at[b2d.at[fs]], ssems.at[fs]).wait()

        def flush(scnt, fs):
            for c in range(_FLUSH // _L):
                off = c * _L
                bv = bflat[pl.ds(off, _L)]
                bv = jnp.where(off + lane < scnt, bv, _DUMP)
                b2d[fs, pl.ds(off, _L)] = bv
            pltpu.async_copy(cols.at[fs], out_h.at[b2d.at[fs]], ssems.at[fs])

        for r in range(_RING):
            @pl.when(r < pcnt)
            def _(r=r):
                fire(pstart + r, r)

        def panel_body(pi, carry):
            scnt, flushed = carry
            p = pstart + pi
            slot = lax.rem(pi, _RING)

            plo = _panel_lo(p)
            plo_rel = plo - wlo
            plo_rel_lo = plo_rel << 14
            plo_rel_hi = (plo_rel + _PW) << 14
            pltpu.make_async_copy(
                tbl_h.at[:, pl.ds(pl.multiple_of(plo, 128), _PW)],
                panels.at[slot], psems.at[slot]).wait()

            def chunk_at(off, carry2):
                scnt2, flushed2 = carry2
                lv = listref[pl.ds(off, _L)]
                m = (lv >= plo_rel_lo) & (lv < plo_rel_hi)
                mc = plsc.all_reduce_population_count(m)[0]

                need_spill = scnt2 + mc > _FLUSH

                @pl.when(need_spill)
                def _():
                    flush(scnt2, lax.rem(flushed2, 2))

                    @pl.when(flushed2 >= 1)
                    def _():
                        drain_flush(lax.rem(flushed2 + 1, 2))

                flushed2 = flushed2 + jnp.where(need_spill, 1, 0)
                scnt2 = jnp.where(need_spill, 0, scnt2)
                active = lax.rem(flushed2, 2)

                @pl.when(mc > 0)
                def _():
                    plsc.store_compressed(tmpu.at[:],
                                          (lv >> 14) - plo_rel, mask=m)
                    plsc.store_compressed(tmpb.at[:], lv & 0x3FFF, mask=m)
                    tu = tmpu[...]
                    tb = tmpb[...]

                    def pair_body(i, _):
                        usp = _splat_lane(tu, i)
                        row = scnt2 + i
                        slotv = jnp.broadcast_to(slot, (_L,))
                        for c in range(_K // _L):
                            kv = c * _L + lane
                            col = plsc.load_gather(panels, [slotv, kv, usp])
                            cols[active, row, pl.ds(c * _L, _L)] = col
                        return 0

                    lax.fori_loop(0, mc, pair_body, 0)
                    plsc.store_compressed(
                        bflat.at[pl.ds(scnt2, _L)], tb, mask=lane < mc)

                return scnt2 + mc, flushed2

            def chunk_body(j, carry2):
                off = pl.multiple_of(j * 2 * _L, _L)
                carry2 = chunk_at(off, carry2)
                return chunk_at(off + _L, carry2)

            carry = lax.fori_loop(0, npair, chunk_body, (scnt, flushed))

            @pl.when(pi + _RING < pcnt)
            def _():
                fire(p + _RING, slot)

            return carry

        scnt, flushed = lax.fori_loop(
            0, pcnt, panel_body, (jnp.int32(0), jnp.int32(0)))

        @pl.when(flushed >= 1)
        def _():
            drain_flush(lax.rem(flushed + 1, 2))
        flush(scnt, lax.rem(flushed, 2))
        drain_flush(lax.rem(flushed, 2))


_mf_gather = functools.partial(
    pl.kernel,
    out_type=(jax.ShapeDtypeStruct((_GROWS, _GK), jnp.float32),
              jax.ShapeDtypeStruct((_GROWS, _GK), jnp.float32)),
    mesh=plsc.VectorSubcoreMesh(core_axis_name="c", subcore_axis_name="s"),
    compiler_params=pltpu.CompilerParams(needs_layout_passes=False),
    scratch_types=[
        pltpu.VMEM((2, _SEC), jnp.int32),
        pltpu.VMEM((_B + 2 * _L,), jnp.int32),
        pltpu.VMEM((_B + 2 * _L,), jnp.int32),
        pltpu.VMEM((_RING, _K, _PW), jnp.float32),
        pltpu.VMEM((2, _FLUSH, _GK), jnp.float32),
        pltpu.VMEM((_FLUSH + _L,), jnp.int32),
        pltpu.VMEM((2, _FLUSH), jnp.int32),
        pltpu.VMEM((_L,), jnp.int32),
        pltpu.VMEM((_L,), jnp.int32),
        pltpu.SemaphoreType.DMA((_RING,)),
        pltpu.SemaphoreType.DMA((2,)),
        pltpu.SemaphoreType.DMA((2,)),
    ],
)(_gather_body)


def _loss_body(user_h, item_h, rating_h, qg_h, pg_h, bu_h, bi_h, out_h,
               uidx, iidx, qrows, prows, bu, bi, rat, stage, sems):
    wid = lax.axis_index("s") * _NC + lax.axis_index("c")
    base = pl.multiple_of(wid * _BPW, _BPW)
    lane = lax.iota(jnp.int32, _L)

    def fire_rows(g, slot):
        rb = pl.multiple_of(base + g * 128, 128)
        return [
            pltpu.async_copy(qg_h.at[pl.ds(rb, 128)], qrows.at[slot],
                             sems.at[0]),
            pltpu.async_copy(pg_h.at[pl.ds(rb, 128)], prows.at[slot],
                             sems.at[1]),
        ]

    cps = []
    row_cps = {0: fire_rows(0, 0)}
    for g in range(4):
        pltpu.sync_copy(user_h.at[pl.ds(base + g * 128, 128)], uidx.at[g])
        pltpu.sync_copy(item_h.at[pl.ds(base + g * 128, 128)], iidx.at[g])
    pltpu.sync_copy(rating_h.at[pl.ds(base, _BPW)], rat)

    for ref, n in ((uidx, _N), (iidx, _N)):
        for g in range(4):
            for c in range(128 // _L):
                v = ref[g, pl.ds(c * _L, _L)] - 1
                ref[g, pl.ds(c * _L, _L)] = jnp.where(v < 0, n - 1, v)

    for g in range(4):
        dst = pl.ds(g * 128, 128)
        cps.append(pltpu.async_copy(bu_h.at[uidx.at[g]], bu.at[dst],
                                    sems.at[2]))
        cps.append(pltpu.async_copy(bi_h.at[iidx.at[g]], bi.at[dst],
                                    sems.at[2]))
    for cp in cps:
        cp.wait()

    sse = jnp.zeros((_L,), jnp.float32)
    for g in range(4):
        slot = g % 2
        if g + 1 < 4:
            row_cps[g + 1] = fire_rows(g + 1, (g + 1) % 2)
        for cp in row_cps.pop(g):
            cp.wait()

        def block_body(b, sse, g=g, slot=slot):
            rb = pl.multiple_of(b * _L, _L)
            dv = jnp.zeros((_L,), jnp.float32)
            for l in range(_L):
                r = rb + l
                acc = (qrows[slot, r, pl.ds(0, _L)]
                       * prows[slot, r, pl.ds(0, _L)])
                for c in range(1, _K // _L):
                    acc = acc + (qrows[slot, r, pl.ds(c * _L, _L)]
                                 * prows[slot, r, pl.ds(c * _L, _L)])
                dv = jnp.where(lane == l, jnp.sum(acc), dv)
            gb = pl.multiple_of(g * 128 + rb, _L)
            ev = (dv + bu[pl.ds(gb, _L)] + bi[pl.ds(gb, _L)]
                  + _RATING_MEAN - rat[pl.ds(gb, _L)])
            return sse + ev * ev

        sse = lax.fori_loop(0, 128 // _L, block_body, sse)
    stage[...] = sse
    pltpu.sync_copy(stage, out_h.at[pl.ds(wid * _L, _L)])


_mf_loss = functools.partial(
    pl.kernel,
    out_type=jax.ShapeDtypeStruct((_NW * _L,), jnp.float32),
    mesh=plsc.VectorSubcoreMesh(core_axis_name="c", subcore_axis_name="s"),
    compiler_params=pltpu.CompilerParams(needs_layout_passes=False),
    scratch_types=[
        pltpu.VMEM((4, 128), jnp.int32),
        pltpu.VMEM((4, 128), jnp.int32),
        pltpu.VMEM((2, 128, _GK), jnp.float32),
        pltpu.VMEM((2, 128, _GK), jnp.float32),
        pltpu.VMEM((_BPW,), jnp.float32),
        pltpu.VMEM((_BPW,), jnp.float32),
        pltpu.VMEM((_BPW,), jnp.float32),
        pltpu.VMEM((_L,), jnp.float32),
        pltpu.SemaphoreType.DMA((3,)),
    ],
)(_loss_body)


@jax.jit
def kernel(user, item, rating, Q, P, bias_users, bias_items):
    qg, pg = _mf_gather(user, item, Q.T, P.T)
    partials = _mf_loss(user, item, rating, qg, pg, bias_users, bias_items)
    return jnp.sum(partials) / _B


# speedup vs baseline: 1.0010x; 1.0010x over previous
import functools

import jax
import jax.numpy as jnp
from jax import lax
from jax.experimental import pallas as pl
from jax.experimental.pallas import tpu as pltpu
from jax.experimental.pallas import tpu_sc as plsc

_N = 1_000_000
_K = 64
_B = 16384
_RATING_MEAN = 3.5

_NC = 2
_NS = 16
_L = 16
_NW = _NC * _NS
_BPW = _B // _NW

_PW = 384
_NLANE = 1_000_064
_NPAN = 2605
_LASTP = _NPAN - 1
_LASTLO = _NLANE - _PW
_PPW = _NPAN // _NW
_PEXTRA = _NPAN - _PPW * _NW
_DUMP = _B
_GROWS = _B + _L
_GK = 128
_FLUSH = 64
_RING = 3
_SEC = 2048


def _panel_lo(p):
    return jnp.where(p >= _LASTP, _LASTLO, p * _PW)


def _splat_lane(v, i):
    idx = jnp.broadcast_to(i.astype(jnp.int32), (_L,))[:, None]
    return lax.gather(
        v, idx,
        dimension_numbers=lax.GatherDimensionNumbers(
            offset_dims=(), collapsed_slice_dims=(0,), start_index_map=(0,)),
        slice_sizes=(1,), mode=lax.GatherScatterMode.PROMISE_IN_BOUNDS)


def _gather_body(user_h, item_h, qt_h, pt_h, qg_h, pg_h,
                 all_idx, listq, listp, panels, cols, bflat, b2d, tmpu, tmpb,
                 psems, ssems, asems):
    wid = lax.axis_index("s") * _NC + lax.axis_index("c")
    pstart = wid * _PPW + jnp.minimum(wid, _PEXTRA)
    pcnt = _PPW + jnp.where(wid < _PEXTRA, 1, 0)
    wlo = _panel_lo(pstart)
    last_p = pstart + pcnt - 1
    whi = jnp.where(last_p >= _LASTP, _NLANE, (last_p + 1) * _PW)
    lane = lax.iota(jnp.int32, _L)

    cnts = []
    for listref, idx_h in ((listq, user_h), (listp, item_h)):
        def fire_sec(s, slot, idx_h=idx_h):
            return pltpu.async_copy(
                idx_h.at[pl.ds(s * _SEC, _SEC)], all_idx.at[slot],
                asems.at[slot])

        cnt = jnp.int32(0)
        fire_sec(0, 0)
        for s in range(_B // _SEC):
            slot = s % 2
            if s + 1 < _B // _SEC:
                fire_sec(s + 1, (s + 1) % 2)
            pltpu.make_async_copy(
                idx_h.at[pl.ds(s * _SEC, _SEC)], all_idx.at[slot],
                asems.at[slot]).wait()

            def scan_body(c, cnt, s=s, slot=slot, listref=listref):
                off = pl.multiple_of(c * _L, _L)
                v = all_idx[slot, pl.ds(off, _L)] - 1
                v = jnp.where(v < 0, _N - 1, v)
                m = (v >= wlo) & (v < whi)
                packed = ((v - wlo) << 14) | (s * _SEC + off + lane)
                plsc.store_compressed(listref.at[pl.ds(cnt, _L)], packed,
                                      mask=m)
                return cnt + plsc.all_reduce_population_count(m)[0]

            cnt = lax.fori_loop(0, _SEC // _L, scan_body, cnt, unroll=2)
        listref[pl.ds(cnt, _L)] = jnp.full((_L,), jnp.int32(0x3FFFFFFF))
        listref[pl.ds(cnt + _L, _L)] = jnp.full((_L,), jnp.int32(0x3FFFFFFF))
        cnts.append(cnt)

    for tbl_h, out_h, listref, cnt in ((qt_h, qg_h, listq, cnts[0]),
                                       (pt_h, pg_h, listp, cnts[1])):
        npair = (cnt + 2 * _L - 1) // (2 * _L)

        def fire(p, slot):
            lo = pl.multiple_of(_panel_lo(p), 128)
            for q in range(4):
                pltpu.async_copy(
                    tbl_h.at[pl.ds(q * _K // 4, _K // 4), pl.ds(lo, _PW)],
                    panels.at[slot, pl.ds(q * _K // 4, _K // 4)],
                    psems.at[slot])

        def drain_flush(fs):
            pltpu.make_async_copy(
                cols.at[fs], out_h.at[b2d.at[fs]], ssems.at[fs]).wait()

        def flush(scnt, fs):
            for c in range(_FLUSH // _L):
                off = c * _L
                bv = bflat[pl.ds(off, _L)]
                bv = jnp.where(off + lane < scnt, bv, _DUMP)
                b2d[fs, pl.ds(off, _L)] = bv
            pltpu.async_copy(cols.at[fs], out_h.at[b2d.at[fs]], ssems.at[fs])

        for r in range(_RING):
            @pl.when(r < pcnt)
            def _(r=r):
                fire(pstart + r, r)

        def panel_body(pi, carry):
            scnt, flushed = carry
            p = pstart + pi
            slot = lax.rem(pi, _RING)

            plo = _panel_lo(p)
            plo_rel = plo - wlo
            plo_rel_lo = plo_rel << 14
            plo_rel_hi = (plo_rel + _PW) << 14
            pltpu.make_async_copy(
                tbl_h.at[:, pl.ds(pl.multiple_of(plo, 128), _PW)],
                panels.at[slot], psems.at[slot]).wait()

            def chunk_at(off, carry2):
                scnt2, flushed2 = carry2
                lv = listref[pl.ds(off, _L)]
                m = (lv >= plo_rel_lo) & (lv < plo_rel_hi)
                mc = plsc.all_reduce_population_count(m)[0]

                need_spill = scnt2 + mc > _FLUSH

                @pl.when(need_spill)
                def _():
                    flush(scnt2, lax.rem(flushed2, 2))

                    @pl.when(flushed2 >= 1)
                    def _():
                        drain_flush(lax.rem(flushed2 + 1, 2))

                flushed2 = flushed2 + jnp.where(need_spill, 1, 0)
                scnt2 = jnp.where(need_spill, 0, scnt2)
                active = lax.rem(flushed2, 2)

                @pl.when(mc > 0)
                def _():
                    plsc.store_compressed(tmpu.at[:],
                                          (lv >> 14) - plo_rel, mask=m)
                    plsc.store_compressed(tmpb.at[:], lv & 0x3FFF, mask=m)
                    tu = tmpu[...]
                    tb = tmpb[...]

                    def pair_body(i, _):
                        usp = _splat_lane(tu, i)
                        row = scnt2 + i
                        slotv = jnp.broadcast_to(slot, (_L,))
                        for c in range(_K // _L):
                            kv = c * _L + lane
                            col = plsc.load_gather(panels, [slotv, kv, usp])
                            cols[active, row, pl.ds(c * _L, _L)] = col
                        return 0

                    lax.fori_loop(0, mc, pair_body, 0)
                    plsc.store_compressed(
                        bflat.at[pl.ds(scnt2, _L)], tb, mask=lane < mc)

                return scnt2 + mc, flushed2

            def chunk_body(j, carry2):
                off = pl.multiple_of(j * 2 * _L, _L)
                carry2 = chunk_at(off, carry2)
                return chunk_at(off + _L, carry2)

            carry = lax.fori_loop(0, npair, chunk_body, (scnt, flushed))

            @pl.when(pi + _RING < pcnt)
            def _():
                fire(p + _RING, slot)

            return carry

        scnt, flushed = lax.fori_loop(
            0, pcnt, panel_body, (jnp.int32(0), jnp.int32(0)))

        @pl.when(flushed >= 1)
        def _():
            drain_flush(lax.rem(flushed + 1, 2))
        flush(scnt, lax.rem(flushed, 2))
        drain_flush(lax.rem(flushed, 2))


_mf_gather = functools.partial(
    pl.kernel,
    out_type=(jax.ShapeDtypeStruct((_GROWS, _GK), jnp.float32),
              jax.ShapeDtypeStruct((_GROWS, _GK), jnp.float32)),
    mesh=plsc.VectorSubcoreMesh(core_axis_name="c", subcore_axis_name="s"),
    compiler_params=pltpu.CompilerParams(needs_layout_passes=False),
    scratch_types=[
        pltpu.VMEM((2, _SEC), jnp.int32),
        pltpu.VMEM((_B + 2 * _L,), jnp.int32),
        pltpu.VMEM((_B + 2 * _L,), jnp.int32),
        pltpu.VMEM((_RING, _K, _PW), jnp.float32),
        pltpu.VMEM((2, _FLUSH, _GK), jnp.float32),
        pltpu.VMEM((_FLUSH + _L,), jnp.int32),
        pltpu.VMEM((2, _FLUSH), jnp.int32),
        pltpu.VMEM((_L,), jnp.int32),
        pltpu.VMEM((_L,), jnp.int32),
        pltpu.SemaphoreType.DMA((_RING,)),
        pltpu.SemaphoreType.DMA((2,)),
        pltpu.SemaphoreType.DMA((2,)),
    ],
)(_gather_body)


def _loss_body(user_h, item_h, rating_h, qg_h, pg_h, bu_h, bi_h, out_h,
               uidx, iidx, qrows, prows, bu, bi, rat, stage, sems):
    wid = lax.axis_index("s") * _NC + lax.axis_index("c")
    base = pl.multiple_of(wid * _BPW, _BPW)
    lane = lax.iota(jnp.int32, _L)

    def fire_rows(g, slot):
        rb = pl.multiple_of(base + g * 128, 128)
        return [
            pltpu.async_copy(qg_h.at[pl.ds(rb, 128)], qrows.at[slot],
                             sems.at[0]),
            pltpu.async_copy(pg_h.at[pl.ds(rb, 128)], prows.at[slot],
                             sems.at[1]),
        ]

    cps = []
    row_cps = {0: fire_rows(0, 0)}
    for g in range(4):
        pltpu.sync_copy(user_h.at[pl.ds(base + g * 128, 128)], uidx.at[g])
        pltpu.sync_copy(item_h.at[pl.ds(base + g * 128, 128)], iidx.at[g])
    pltpu.sync_copy(rating_h.at[pl.ds(base, _BPW)], rat)

    for ref, n in ((uidx, _N), (iidx, _N)):
        for g in range(4):
            for c in range(128 // _L):
                v = ref[g, pl.ds(c * _L, _L)] - 1
                ref[g, pl.ds(c * _L, _L)] = jnp.where(v < 0, n - 1, v)

    for g in range(4):
        dst = pl.ds(g * 128, 128)
        cps.append(pltpu.async_copy(bu_h.at[uidx.at[g]], bu.at[dst],
                                    sems.at[2]))
        cps.append(pltpu.async_copy(bi_h.at[iidx.at[g]], bi.at[dst],
                                    sems.at[2]))
    for cp in cps:
        cp.wait()

    sse = jnp.zeros((_L,), jnp.float32)
    for g in range(4):
        slot = g % 2
        if g + 1 < 4:
            row_cps[g + 1] = fire_rows(g + 1, (g + 1) % 2)
        for cp in row_cps.pop(g):
            cp.wait()

        def block_body(b, sse, g=g, slot=slot):
            rb = pl.multiple_of(b * _L, _L)
            dv = jnp.zeros((_L,), jnp.float32)
            for l in range(_L):
                r = rb + l
                acc = (qrows[slot, r, pl.ds(0, _L)]
                       * prows[slot, r, pl.ds(0, _L)])
                for c in range(1, _K // _L):
                    acc = acc + (qrows[slot, r, pl.ds(c * _L, _L)]
                                 * prows[slot, r, pl.ds(c * _L, _L)])
                dv = jnp.where(lane == l, jnp.sum(acc), dv)
            gb = pl.multiple_of(g * 128 + rb, _L)
            ev = (dv + bu[pl.ds(gb, _L)] + bi[pl.ds(gb, _L)]
                  + _RATING_MEAN - rat[pl.ds(gb, _L)])
            return sse + ev * ev

        sse = lax.fori_loop(0, 128 // _L, block_body, sse)
    stage[...] = sse
    pltpu.sync_copy(stage, out_h.at[pl.ds(wid * _L, _L)])


_mf_loss = functools.partial(
    pl.kernel,
    out_type=jax.ShapeDtypeStruct((_NW * _L,), jnp.float32),
    mesh=plsc.VectorSubcoreMesh(core_axis_name="c", subcore_axis_name="s"),
    compiler_params=pltpu.CompilerParams(needs_layout_passes=False),
    scratch_types=[
        pltpu.VMEM((4, 128), jnp.int32),
        pltpu.VMEM((4, 128), jnp.int32),
        pltpu.VMEM((2, 128, _GK), jnp.float32),
        pltpu.VMEM((2, 128, _GK), jnp.float32),
        pltpu.VMEM((_BPW,), jnp.float32),
        pltpu.VMEM((_BPW,), jnp.float32),
        pltpu.VMEM((_BPW,), jnp.float32),
        pltpu.VMEM((_L,), jnp.float32),
        pltpu.SemaphoreType.DMA((3,)),
    ],
)(_loss_body)


@jax.jit
def kernel(user, item, rating, Q, P, bias_users, bias_items):
    qg, pg = _mf_gather(user, item, Q.T, P.T)
    partials = _mf_loss(user, item, rating, qg, pg, bias_users, bias_items)
    return jnp.sum(partials) / _B


# speedup vs baseline: 1.0268x; 1.0258x over previous
import functools

import jax
import jax.numpy as jnp
from jax import lax
from jax.experimental import pallas as pl
from jax.experimental.pallas import tpu as pltpu
from jax.experimental.pallas import tpu_sc as plsc

_N = 1_000_000
_K = 64
_B = 16384
_RATING_MEAN = 3.5

_NC = 2
_NS = 16
_L = 16
_NW = _NC * _NS
_BPW = _B // _NW

_PW = 384
_NLANE = 1_000_064
_NPAN = 2605
_LASTP = _NPAN - 1
_LASTLO = _NLANE - _PW
_PPW = _NPAN // _NW
_PEXTRA = _NPAN - _PPW * _NW
_DUMP = _B
_GROWS = _B + _L
_GK = 128
_FLUSH = 64
_RING = 3
_SEC = 2048


def _panel_lo(p):
    return jnp.where(p >= _LASTP, _LASTLO, p * _PW)


def _splat_lane(v, i):
    idx = jnp.broadcast_to(i.astype(jnp.int32), (_L,))[:, None]
    return lax.gather(
        v, idx,
        dimension_numbers=lax.GatherDimensionNumbers(
            offset_dims=(), collapsed_slice_dims=(0,), start_index_map=(0,)),
        slice_sizes=(1,), mode=lax.GatherScatterMode.PROMISE_IN_BOUNDS)


def _gather_body(user_h, item_h, qt_h, pt_h, qg_h, pg_h,
                 all_idx, listq, listp, panels, cols, bflat, b2d, tmpu, tmpb,
                 psems, ssems, asems):
    wid = lax.axis_index("s") * _NC + lax.axis_index("c")
    pstart = wid * _PPW + jnp.minimum(wid, _PEXTRA)
    pcnt = _PPW + jnp.where(wid < _PEXTRA, 1, 0)
    wlo = _panel_lo(pstart)
    last_p = pstart + pcnt - 1
    whi = jnp.where(last_p >= _LASTP, _NLANE, (last_p + 1) * _PW)
    lane = lax.iota(jnp.int32, _L)

    for tbl_h, out_h, idx_h in ((qt_h, qg_h, user_h), (pt_h, pg_h, item_h)):
        def fire_sec(s, slot, idx_h=idx_h):
            return pltpu.async_copy(
                idx_h.at[pl.ds(s * _SEC, _SEC)], all_idx.at[slot],
                asems.at[slot])

        cnt = jnp.int32(0)
        fire_sec(0, 0)
        for s in range(_B // _SEC):
            slot = s % 2
            if s + 1 < _B // _SEC:
                fire_sec(s + 1, (s + 1) % 2)
            pltpu.make_async_copy(
                idx_h.at[pl.ds(s * _SEC, _SEC)], all_idx.at[slot],
                asems.at[slot]).wait()

            def scan_body(c, cnt, s=s, slot=slot):
                off = pl.multiple_of(c * _L, _L)
                v = all_idx[slot, pl.ds(off, _L)] - 1
                v = jnp.where(v < 0, _N - 1, v)
                m = (v >= wlo) & (v < whi)
                plsc.store_compressed(listq.at[pl.ds(cnt, _L)], v, mask=m)
                plsc.store_compressed(listp.at[pl.ds(cnt, _L)],
                                      s * _SEC + off + lane, mask=m)
                return cnt + plsc.all_reduce_population_count(m)[0]

            cnt = lax.fori_loop(0, _SEC // _L, scan_body, cnt, unroll=2)
        listq[pl.ds(cnt, _L)] = jnp.full((_L,), jnp.int32(0x3FFFFFFF))
        listq[pl.ds(cnt + _L, _L)] = jnp.full((_L,), jnp.int32(0x3FFFFFFF))
        npair = (cnt + 2 * _L - 1) // (2 * _L)

        def fire(p, slot):
            lo = pl.multiple_of(_panel_lo(p), 128)
            for q in range(4):
                pltpu.async_copy(
                    tbl_h.at[pl.ds(q * _K // 4, _K // 4), pl.ds(lo, _PW)],
                    panels.at[slot, pl.ds(q * _K // 4, _K // 4)],
                    psems.at[slot])

        def drain_flush(fs):
            pltpu.make_async_copy(
                cols.at[fs], out_h.at[b2d.at[fs]], ssems.at[fs]).wait()

        def flush(scnt, fs):
            for c in range(_FLUSH // _L):
                off = c * _L
                bv = bflat[pl.ds(off, _L)]
                bv = jnp.where(off + lane < scnt, bv, _DUMP)
                b2d[fs, pl.ds(off, _L)] = bv
            pltpu.async_copy(cols.at[fs], out_h.at[b2d.at[fs]], ssems.at[fs])

        for r in range(_RING):
            @pl.when(r < pcnt)
            def _(r=r):
                fire(pstart + r, r)

        def panel_body(pi, carry):
            scnt, flushed = carry
            p = pstart + pi
            slot = lax.rem(pi, _RING)

            plo = _panel_lo(p)
            pltpu.make_async_copy(
                tbl_h.at[:, pl.ds(pl.multiple_of(plo, 128), _PW)],
                panels.at[slot], psems.at[slot]).wait()

            def chunk_at(off, carry2):
                scnt2, flushed2 = carry2
                lv = listq[pl.ds(off, _L)]
                m = (lv >= plo) & (lv < plo + _PW)
                mc = plsc.all_reduce_population_count(m)[0]

                need_spill = scnt2 + mc > _FLUSH

                @pl.when(need_spill)
                def _():
                    flush(scnt2, lax.rem(flushed2, 2))

                    @pl.when(flushed2 >= 1)
                    def _():
                        drain_flush(lax.rem(flushed2 + 1, 2))

                flushed2 = flushed2 + jnp.where(need_spill, 1, 0)
                scnt2 = jnp.where(need_spill, 0, scnt2)
                active = lax.rem(flushed2, 2)

                @pl.when(mc > 0)
                def _():
                    bv = listp[pl.ds(off, _L)]
                    plsc.store_compressed(tmpu.at[:], lv - plo, mask=m)
                    plsc.store_compressed(tmpb.at[:], bv, mask=m)
                    tu = tmpu[...]
                    tb = tmpb[...]

                    def pair_body(i, _):
                        usp = _splat_lane(tu, i)
                        row = scnt2 + i
                        slotv = jnp.broadcast_to(slot, (_L,))
                        for c in range(_K // _L):
                            kv = c * _L + lane
                            col = plsc.load_gather(panels, [slotv, kv, usp])
                            cols[active, row, pl.ds(c * _L, _L)] = col
                        return 0

                    lax.fori_loop(0, mc, pair_body, 0)
                    plsc.store_compressed(
                        bflat.at[pl.ds(scnt2, _L)], tb, mask=lane < mc)

                return scnt2 + mc, flushed2

            def chunk_body(j, carry2):
                off = pl.multiple_of(j * 2 * _L, _L)
                carry2 = chunk_at(off, carry2)
                return chunk_at(off + _L, carry2)

            carry = lax.fori_loop(0, npair, chunk_body, (scnt, flushed))

            @pl.when(pi + _RING < pcnt)
            def _():
                fire(p + _RING, slot)

            return carry

        scnt, flushed = lax.fori_loop(
            0, pcnt, panel_body, (jnp.int32(0), jnp.int32(0)))

        @pl.when(flushed >= 1)
        def _():
            drain_flush(lax.rem(flushed + 1, 2))
        flush(scnt, lax.rem(flushed, 2))
        drain_flush(lax.rem(flushed, 2))


_mf_gather = functools.partial(
    pl.kernel,
    out_type=(jax.ShapeDtypeStruct((_GROWS, _GK), jnp.float32),
              jax.ShapeDtypeStruct((_GROWS, _GK), jnp.float32)),
    mesh=plsc.VectorSubcoreMesh(core_axis_name="c", subcore_axis_name="s"),
    compiler_params=pltpu.CompilerParams(needs_layout_passes=False),
    scratch_types=[
        pltpu.VMEM((2, _SEC), jnp.int32),
        pltpu.VMEM((_B + 2 * _L,), jnp.int32),
        pltpu.VMEM((_B + 2 * _L,), jnp.int32),
        pltpu.VMEM((_RING, _K, _PW), jnp.float32),
        pltpu.VMEM((2, _FLUSH, _GK), jnp.float32),
        pltpu.VMEM((_FLUSH + _L,), jnp.int32),
        pltpu.VMEM((2, _FLUSH), jnp.int32),
        pltpu.VMEM((_L,), jnp.int32),
        pltpu.VMEM((_L,), jnp.int32),
        pltpu.SemaphoreType.DMA((_RING,)),
        pltpu.SemaphoreType.DMA((2,)),
        pltpu.SemaphoreType.DMA((2,)),
    ],
)(_gather_body)


def _loss_body(user_h, item_h, rating_h, qg_h, pg_h, bu_h, bi_h, out_h,
               uidx, iidx, qrows, prows, bu, bi, rat, stage, sems):
    wid = lax.axis_index("s") * _NC + lax.axis_index("c")
    base = pl.multiple_of(wid * _BPW, _BPW)
    lane = lax.iota(jnp.int32, _L)

    def fire_rows(g, slot):
        rb = pl.multiple_of(base + g * 128, 128)
        return [
            pltpu.async_copy(qg_h.at[pl.ds(rb, 128)], qrows.at[slot],
                             sems.at[0]),
            pltpu.async_copy(pg_h.at[pl.ds(rb, 128)], prows.at[slot],
                             sems.at[1]),
        ]

    cps = []
    row_cps = {0: fire_rows(0, 0)}
    for g in range(4):
        pltpu.sync_copy(user_h.at[pl.ds(base + g * 128, 128)], uidx.at[g])
        pltpu.sync_copy(item_h.at[pl.ds(base + g * 128, 128)], iidx.at[g])
    pltpu.sync_copy(rating_h.at[pl.ds(base, _BPW)], rat)

    for ref, n in ((uidx, _N), (iidx, _N)):
        for g in range(4):
            for c in range(128 // _L):
                v = ref[g, pl.ds(c * _L, _L)] - 1
                ref[g, pl.ds(c * _L, _L)] = jnp.where(v < 0, n - 1, v)

    for g in range(4):
        dst = pl.ds(g * 128, 128)
        cps.append(pltpu.async_copy(bu_h.at[uidx.at[g]], bu.at[dst],
                                    sems.at[2]))
        cps.append(pltpu.async_copy(bi_h.at[iidx.at[g]], bi.at[dst],
                                    sems.at[2]))
    for cp in cps:
        cp.wait()

    sse = jnp.zeros((_L,), jnp.float32)
    for g in range(4):
        slot = g % 2
        if g + 1 < 4:
            row_cps[g + 1] = fire_rows(g + 1, (g + 1) % 2)
        for cp in row_cps.pop(g):
            cp.wait()

        def block_body(b, sse, g=g, slot=slot):
            rb = pl.multiple_of(b * _L, _L)
            dv = jnp.zeros((_L,), jnp.float32)
            for l in range(_L):
                r = rb + l
                acc = (qrows[slot, r, pl.ds(0, _L)]
                       * prows[slot, r, pl.ds(0, _L)])
                for c in range(1, _K // _L):
                    acc = acc + (qrows[slot, r, pl.ds(c * _L, _L)]
                                 * prows[slot, r, pl.ds(c * _L, _L)])
                dv = jnp.where(lane == l, jnp.sum(acc), dv)
            gb = pl.multiple_of(g * 128 + rb, _L)
            ev = (dv + bu[pl.ds(gb, _L)] + bi[pl.ds(gb, _L)]
                  + _RATING_MEAN - rat[pl.ds(gb, _L)])
            return sse + ev * ev

        sse = lax.fori_loop(0, 128 // _L, block_body, sse)
    stage[...] = sse
    pltpu.sync_copy(stage, out_h.at[pl.ds(wid * _L, _L)])


_mf_loss = functools.partial(
    pl.kernel,
    out_type=jax.ShapeDtypeStruct((_NW * _L,), jnp.float32),
    mesh=plsc.VectorSubcoreMesh(core_axis_name="c", subcore_axis_name="s"),
    compiler_params=pltpu.CompilerParams(needs_layout_passes=False),
    scratch_types=[
        pltpu.VMEM((4, 128), jnp.int32),
        pltpu.VMEM((4, 128), jnp.int32),
        pltpu.VMEM((2, 128, _GK), jnp.float32),
        pltpu.VMEM((2, 128, _GK), jnp.float32),
        pltpu.VMEM((_BPW,), jnp.float32),
        pltpu.VMEM((_BPW,), jnp.float32),
        pltpu.VMEM((_BPW,), jnp.float32),
        pltpu.VMEM((_L,), jnp.float32),
        pltpu.SemaphoreType.DMA((3,)),
    ],
)(_loss_body)


@jax.jit
def kernel(user, item, rating, Q, P, bias_users, bias_items):
    qg, pg = _mf_gather(user, item, Q.T, P.T)
    partials = _mf_loss(user, item, rating, qg, pg, bias_users, bias_items)
    return jnp.sum(partials) / _B


# speedup vs baseline: 1.1477x; 1.1177x over previous
import functools

import jax
import jax.numpy as jnp
from jax import lax
from jax.experimental import pallas as pl
from jax.experimental.pallas import tpu as pltpu
from jax.experimental.pallas import tpu_sc as plsc

_N = 1_000_000
_K = 64
_B = 16384
_RATING_MEAN = 3.5

_NC = 2
_NS = 16
_L = 16
_NW = _NC * _NS
_BPW = _B // _NW

_PW = 512
_NLANE = 1_000_064
_NPAN = 1954
_LASTP = _NPAN - 1
_LASTLO = _NLANE - _PW
_PPW = _NPAN // _NW
_PEXTRA = _NPAN - _PPW * _NW
_DUMP = _B
_GROWS = _B + _L
_GK = 128
_FLUSH = 64
_RING = 2
_SEC = 2048


def _panel_lo(p):
    return jnp.where(p >= _LASTP, _LASTLO, p * _PW)


def _splat_lane(v, i):
    idx = jnp.broadcast_to(i.astype(jnp.int32), (_L,))[:, None]
    return lax.gather(
        v, idx,
        dimension_numbers=lax.GatherDimensionNumbers(
            offset_dims=(), collapsed_slice_dims=(0,), start_index_map=(0,)),
        slice_sizes=(1,), mode=lax.GatherScatterMode.PROMISE_IN_BOUNDS)


def _gather_body(user_h, item_h, qt_h, pt_h, qg_h, pg_h,
                 all_idx, listq, listp, panels, cols, bflat, b2d, tmpu, tmpb,
                 psems, ssems, asems):
    wid = lax.axis_index("s") * _NC + lax.axis_index("c")
    pstart = wid * _PPW + jnp.minimum(wid, _PEXTRA)
    pcnt = _PPW + jnp.where(wid < _PEXTRA, 1, 0)
    wlo = _panel_lo(pstart)
    last_p = pstart + pcnt - 1
    whi = jnp.where(last_p >= _LASTP, _NLANE, (last_p + 1) * _PW)
    lane = lax.iota(jnp.int32, _L)

    for tbl_h, out_h, idx_h in ((qt_h, qg_h, user_h), (pt_h, pg_h, item_h)):
        def fire_sec(s, slot, idx_h=idx_h):
            return pltpu.async_copy(
                idx_h.at[pl.ds(s * _SEC, _SEC)], all_idx.at[slot],
                asems.at[slot])

        cnt = jnp.int32(0)
        fire_sec(0, 0)
        for s in range(_B // _SEC):
            slot = s % 2
            if s + 1 < _B // _SEC:
                fire_sec(s + 1, (s + 1) % 2)
            pltpu.make_async_copy(
                idx_h.at[pl.ds(s * _SEC, _SEC)], all_idx.at[slot],
                asems.at[slot]).wait()

            def scan_body(c, cnt, s=s, slot=slot):
                off = pl.multiple_of(c * _L, _L)
                v = all_idx[slot, pl.ds(off, _L)] - 1
                v = jnp.where(v < 0, _N - 1, v)
                m = (v >= wlo) & (v < whi)
                plsc.store_compressed(listq.at[pl.ds(cnt, _L)], v, mask=m)
                plsc.store_compressed(listp.at[pl.ds(cnt, _L)],
                                      s * _SEC + off + lane, mask=m)
                return cnt + plsc.all_reduce_population_count(m)[0]

            cnt = lax.fori_loop(0, _SEC // _L, scan_body, cnt, unroll=2)
        listq[pl.ds(cnt, _L)] = jnp.full((_L,), jnp.int32(0x3FFFFFFF))
        listq[pl.ds(cnt + _L, _L)] = jnp.full((_L,), jnp.int32(0x3FFFFFFF))
        npair = (cnt + 2 * _L - 1) // (2 * _L)

        def fire(p, slot):
            lo = pl.multiple_of(_panel_lo(p), 128)
            for q in range(4):
                pltpu.async_copy(
                    tbl_h.at[pl.ds(q * _K // 4, _K // 4), pl.ds(lo, _PW)],
                    panels.at[slot, pl.ds(q * _K // 4, _K // 4)],
                    psems.at[slot])

        def drain_flush(fs):
            pltpu.make_async_copy(
                cols.at[fs], out_h.at[b2d.at[fs]], ssems.at[fs]).wait()

        def flush(scnt, fs):
            for c in range(_FLUSH // _L):
                off = c * _L
                bv = bflat[pl.ds(off, _L)]
                bv = jnp.where(off + lane < scnt, bv, _DUMP)
                b2d[fs, pl.ds(off, _L)] = bv
            pltpu.async_copy(cols.at[fs], out_h.at[b2d.at[fs]], ssems.at[fs])

        for r in range(_RING):
            @pl.when(r < pcnt)
            def _(r=r):
                fire(pstart + r, r)

        def panel_body(pi, carry):
            scnt, flushed = carry
            p = pstart + pi
            slot = lax.rem(pi, _RING)

            plo = _panel_lo(p)
            pltpu.make_async_copy(
                tbl_h.at[:, pl.ds(pl.multiple_of(plo, 128), _PW)],
                panels.at[slot], psems.at[slot]).wait()

            def chunk_at(off, carry2):
                scnt2, flushed2 = carry2
                lv = listq[pl.ds(off, _L)]
                m = (lv >= plo) & (lv < plo + _PW)
                mc = plsc.all_reduce_population_count(m)[0]

                need_spill = scnt2 + mc > _FLUSH

                @pl.when(need_spill)
                def _():
                    flush(scnt2, lax.rem(flushed2, 2))

                    @pl.when(flushed2 >= 1)
                    def _():
                        drain_flush(lax.rem(flushed2 + 1, 2))

                flushed2 = flushed2 + jnp.where(need_spill, 1, 0)
                scnt2 = jnp.where(need_spill, 0, scnt2)
                active = lax.rem(flushed2, 2)

                @pl.when(mc > 0)
                def _():
                    bv = listp[pl.ds(off, _L)]
                    plsc.store_compressed(tmpu.at[:], lv - plo, mask=m)
                    plsc.store_compressed(tmpb.at[:], bv, mask=m)
                    tu = tmpu[...]
                    tb = tmpb[...]

                    def pair_body(i, _):
                        usp = _splat_lane(tu, i)
                        row = scnt2 + i
                        slotv = jnp.broadcast_to(slot, (_L,))
                        for c in range(_K // _L):
                            kv = c * _L + lane
                            col = plsc.load_gather(panels, [slotv, kv, usp])
                            cols[active, row, pl.ds(c * _L, _L)] = col
                        return 0

                    lax.fori_loop(0, mc, pair_body, 0)
                    plsc.store_compressed(
                        bflat.at[pl.ds(scnt2, _L)], tb, mask=lane < mc)

                return scnt2 + mc, flushed2

            def chunk_body(j, carry2):
                off = pl.multiple_of(j * 2 * _L, _L)
                carry2 = chunk_at(off, carry2)
                return chunk_at(off + _L, carry2)

            carry = lax.fori_loop(0, npair, chunk_body, (scnt, flushed))

            @pl.when(pi + _RING < pcnt)
            def _():
                fire(p + _RING, slot)

            return carry

        scnt, flushed = lax.fori_loop(
            0, pcnt, panel_body, (jnp.int32(0), jnp.int32(0)))

        @pl.when(flushed >= 1)
        def _():
            drain_flush(lax.rem(flushed + 1, 2))
        flush(scnt, lax.rem(flushed, 2))
        drain_flush(lax.rem(flushed, 2))


_mf_gather = functools.partial(
    pl.kernel,
    out_type=(jax.ShapeDtypeStruct((_GROWS, _GK), jnp.float32),
              jax.ShapeDtypeStruct((_GROWS, _GK), jnp.float32)),
    mesh=plsc.VectorSubcoreMesh(core_axis_name="c", subcore_axis_name="s"),
    compiler_params=pltpu.CompilerParams(needs_layout_passes=False),
    scratch_types=[
        pltpu.VMEM((2, _SEC), jnp.int32),
        pltpu.VMEM((_B + 2 * _L,), jnp.int32),
        pltpu.VMEM((_B + 2 * _L,), jnp.int32),
        pltpu.VMEM((_RING, _K, _PW), jnp.float32),
        pltpu.VMEM((2, _FLUSH, _GK), jnp.float32),
        pltpu.VMEM((_FLUSH + _L,), jnp.int32),
        pltpu.VMEM((2, _FLUSH), jnp.int32),
        pltpu.VMEM((_L,), jnp.int32),
        pltpu.VMEM((_L,), jnp.int32),
        pltpu.SemaphoreType.DMA((_RING,)),
        pltpu.SemaphoreType.DMA((2,)),
        pltpu.SemaphoreType.DMA((2,)),
    ],
)(_gather_body)


def _loss_body(user_h, item_h, rating_h, qg_h, pg_h, bu_h, bi_h, out_h,
               uidx, iidx, qrows, prows, bu, bi, rat, stage, sems):
    wid = lax.axis_index("s") * _NC + lax.axis_index("c")
    base = pl.multiple_of(wid * _BPW, _BPW)
    lane = lax.iota(jnp.int32, _L)

    def fire_rows(g, slot):
        rb = pl.multiple_of(base + g * 128, 128)
        return [
            pltpu.async_copy(qg_h.at[pl.ds(rb, 128)], qrows.at[slot],
                             sems.at[0]),
            pltpu.async_copy(pg_h.at[pl.ds(rb, 128)], prows.at[slot],
                             sems.at[1]),
        ]

    cps = []
    row_cps = {0: fire_rows(0, 0)}
    for g in range(4):
        pltpu.sync_copy(user_h.at[pl.ds(base + g * 128, 128)], uidx.at[g])
        pltpu.sync_copy(item_h.at[pl.ds(base + g * 128, 128)], iidx.at[g])
    pltpu.sync_copy(rating_h.at[pl.ds(base, _BPW)], rat)

    for ref, n in ((uidx, _N), (iidx, _N)):
        for g in range(4):
            for c in range(128 // _L):
                v = ref[g, pl.ds(c * _L, _L)] - 1
                ref[g, pl.ds(c * _L, _L)] = jnp.where(v < 0, n - 1, v)

    for g in range(4):
        dst = pl.ds(g * 128, 128)
        cps.append(pltpu.async_copy(bu_h.at[uidx.at[g]], bu.at[dst],
                                    sems.at[2]))
        cps.append(pltpu.async_copy(bi_h.at[iidx.at[g]], bi.at[dst],
                                    sems.at[2]))
    for cp in cps:
        cp.wait()

    sse = jnp.zeros((_L,), jnp.float32)
    for g in range(4):
        slot = g % 2
        if g + 1 < 4:
            row_cps[g + 1] = fire_rows(g + 1, (g + 1) % 2)
        for cp in row_cps.pop(g):
            cp.wait()

        def block_body(b, sse, g=g, slot=slot):
            rb = pl.multiple_of(b * _L, _L)
            dv = jnp.zeros((_L,), jnp.float32)
            for l in range(_L):
                r = rb + l
                acc = (qrows[slot, r, pl.ds(0, _L)]
                       * prows[slot, r, pl.ds(0, _L)])
                for c in range(1, _K // _L):
                    acc = acc + (qrows[slot, r, pl.ds(c * _L, _L)]
                                 * prows[slot, r, pl.ds(c * _L, _L)])
                dv = jnp.where(lane == l, jnp.sum(acc), dv)
            gb = pl.multiple_of(g * 128 + rb, _L)
            ev = (dv + bu[pl.ds(gb, _L)] + bi[pl.ds(gb, _L)]
                  + _RATING_MEAN - rat[pl.ds(gb, _L)])
            return sse + ev * ev

        sse = lax.fori_loop(0, 128 // _L, block_body, sse)
    stage[...] = sse
    pltpu.sync_copy(stage, out_h.at[pl.ds(wid * _L, _L)])


_mf_loss = functools.partial(
    pl.kernel,
    out_type=jax.ShapeDtypeStruct((_NW * _L,), jnp.float32),
    mesh=plsc.VectorSubcoreMesh(core_axis_name="c", subcore_axis_name="s"),
    compiler_params=pltpu.CompilerParams(needs_layout_passes=False),
    scratch_types=[
        pltpu.VMEM((4, 128), jnp.int32),
        pltpu.VMEM((4, 128), jnp.int32),
        pltpu.VMEM((2, 128, _GK), jnp.float32),
        pltpu.VMEM((2, 128, _GK), jnp.float32),
        pltpu.VMEM((_BPW,), jnp.float32),
        pltpu.VMEM((_BPW,), jnp.float32),
        pltpu.VMEM((_BPW,), jnp.float32),
        pltpu.VMEM((_L,), jnp.float32),
        pltpu.SemaphoreType.DMA((3,)),
    ],
)(_loss_body)


@jax.jit
def kernel(user, item, rating, Q, P, bias_users, bias_items):
    qg, pg = _mf_gather(user, item, Q.T, P.T)
    partials = _mf_loss(user, item, rating, qg, pg, bias_users, bias_items)
    return jnp.sum(partials) / _B


# speedup vs baseline: 1.4597x; 1.2718x over previous
import functools

import jax
import jax.numpy as jnp
from jax import lax
from jax.experimental import pallas as pl
from jax.experimental.pallas import tpu as pltpu
from jax.experimental.pallas import tpu_sc as plsc

_N = 1_000_000
_K = 64
_B = 16384
_RATING_MEAN = 3.5

_NC = 2
_NS = 16
_L = 16
_NW = _NC * _NS
_BPW = _B // _NW

_PW = 640
_NLANE = 1_000_064
_NPAN = 1563
_LASTP = _NPAN - 1
_LASTLO = _NLANE - _PW
_PPW = _NPAN // _NW
_PEXTRA = _NPAN - _PPW * _NW
_DUMP = _B
_GROWS = _B + _L
_GK = 128
_FLUSH = 32
_RING = 2
_SEC = 2048


def _panel_lo(p):
    return jnp.where(p >= _LASTP, _LASTLO, p * _PW)


def _splat_lane(v, i):
    idx = jnp.broadcast_to(i.astype(jnp.int32), (_L,))[:, None]
    return lax.gather(
        v, idx,
        dimension_numbers=lax.GatherDimensionNumbers(
            offset_dims=(), collapsed_slice_dims=(0,), start_index_map=(0,)),
        slice_sizes=(1,), mode=lax.GatherScatterMode.PROMISE_IN_BOUNDS)


def _gather_body(user_h, item_h, qt_h, pt_h, qg_h, pg_h,
                 all_idx, listq, listp, panels, cols, bflat, b2d, tmpu, tmpb,
                 psems, ssems, asems):
    wid = lax.axis_index("s") * _NC + lax.axis_index("c")
    pstart = wid * _PPW + jnp.minimum(wid, _PEXTRA)
    pcnt = _PPW + jnp.where(wid < _PEXTRA, 1, 0)
    wlo = _panel_lo(pstart)
    last_p = pstart + pcnt - 1
    whi = jnp.where(last_p >= _LASTP, _NLANE, (last_p + 1) * _PW)
    lane = lax.iota(jnp.int32, _L)

    for tbl_h, out_h, idx_h in ((qt_h, qg_h, user_h), (pt_h, pg_h, item_h)):
        def fire_sec(s, slot, idx_h=idx_h):
            return pltpu.async_copy(
                idx_h.at[pl.ds(s * _SEC, _SEC)], all_idx.at[slot],
                asems.at[slot])

        cnt = jnp.int32(0)
        fire_sec(0, 0)
        for s in range(_B // _SEC):
            slot = s % 2
            if s + 1 < _B // _SEC:
                fire_sec(s + 1, (s + 1) % 2)
            pltpu.make_async_copy(
                idx_h.at[pl.ds(s * _SEC, _SEC)], all_idx.at[slot],
                asems.at[slot]).wait()

            def scan_body(c, cnt, s=s, slot=slot):
                off = pl.multiple_of(c * _L, _L)
                v = all_idx[slot, pl.ds(off, _L)] - 1
                v = jnp.where(v < 0, _N - 1, v)
                m = (v >= wlo) & (v < whi)
                plsc.store_compressed(listq.at[pl.ds(cnt, _L)], v, mask=m)
                plsc.store_compressed(listp.at[pl.ds(cnt, _L)],
                                      s * _SEC + off + lane, mask=m)
                return cnt + plsc.all_reduce_population_count(m)[0]

            cnt = lax.fori_loop(0, _SEC // _L, scan_body, cnt, unroll=2)
        listq[pl.ds(cnt, _L)] = jnp.full((_L,), jnp.int32(0x3FFFFFFF))
        listq[pl.ds(cnt + _L, _L)] = jnp.full((_L,), jnp.int32(0x3FFFFFFF))
        npair = (cnt + 2 * _L - 1) // (2 * _L)

        def fire(p, slot):
            lo = pl.multiple_of(_panel_lo(p), 128)
            for q in range(4):
                pltpu.async_copy(
                    tbl_h.at[pl.ds(q * _K // 4, _K // 4), pl.ds(lo, _PW)],
                    panels.at[slot, pl.ds(q * _K // 4, _K // 4)],
                    psems.at[slot])

        def drain_flush(fs):
            pltpu.make_async_copy(
                cols.at[fs], out_h.at[b2d.at[fs]], ssems.at[fs]).wait()

        def flush(scnt, fs):
            for c in range(_FLUSH // _L):
                off = c * _L
                bv = bflat[pl.ds(off, _L)]
                bv = jnp.where(off + lane < scnt, bv, _DUMP)
                b2d[fs, pl.ds(off, _L)] = bv
            pltpu.async_copy(cols.at[fs], out_h.at[b2d.at[fs]], ssems.at[fs])

        for r in range(_RING):
            @pl.when(r < pcnt)
            def _(r=r):
                fire(pstart + r, r)

        def panel_body(pi, carry):
            scnt, flushed = carry
            p = pstart + pi
            slot = lax.rem(pi, _RING)

            plo = _panel_lo(p)
            pltpu.make_async_copy(
                tbl_h.at[:, pl.ds(pl.multiple_of(plo, 128), _PW)],
                panels.at[slot], psems.at[slot]).wait()

            def chunk_at(off, carry2):
                scnt2, flushed2 = carry2
                lv = listq[pl.ds(off, _L)]
                m = (lv >= plo) & (lv < plo + _PW)
                mc = plsc.all_reduce_population_count(m)[0]

                need_spill = scnt2 + mc > _FLUSH

                @pl.when(need_spill)
                def _():
                    flush(scnt2, lax.rem(flushed2, 2))

                    @pl.when(flushed2 >= 1)
                    def _():
                        drain_flush(lax.rem(flushed2 + 1, 2))

                flushed2 = flushed2 + jnp.where(need_spill, 1, 0)
                scnt2 = jnp.where(need_spill, 0, scnt2)
                active = lax.rem(flushed2, 2)

                @pl.when(mc > 0)
                def _():
                    bv = listp[pl.ds(off, _L)]
                    plsc.store_compressed(tmpu.at[:], lv - plo, mask=m)
                    plsc.store_compressed(tmpb.at[:], bv, mask=m)
                    tu = tmpu[...]
                    tb = tmpb[...]

                    def pair_body(i, _):
                        usp = _splat_lane(tu, i)
                        row = scnt2 + i
                        slotv = jnp.broadcast_to(slot, (_L,))
                        for c in range(_K // _L):
                            kv = c * _L + lane
                            col = plsc.load_gather(panels, [slotv, kv, usp])
                            cols[active, row, pl.ds(c * _L, _L)] = col
                        return 0

                    lax.fori_loop(0, mc, pair_body, 0)
                    plsc.store_compressed(
                        bflat.at[pl.ds(scnt2, _L)], tb, mask=lane < mc)

                return scnt2 + mc, flushed2

            def chunk_body(j, carry2):
                off = pl.multiple_of(j * 2 * _L, _L)
                carry2 = chunk_at(off, carry2)
                return chunk_at(off + _L, carry2)

            carry = lax.fori_loop(0, npair, chunk_body, (scnt, flushed))

            @pl.when(pi + _RING < pcnt)
            def _():
                fire(p + _RING, slot)

            return carry

        scnt, flushed = lax.fori_loop(
            0, pcnt, panel_body, (jnp.int32(0), jnp.int32(0)))

        @pl.when(flushed >= 1)
        def _():
            drain_flush(lax.rem(flushed + 1, 2))
        flush(scnt, lax.rem(flushed, 2))
        drain_flush(lax.rem(flushed, 2))


_mf_gather = functools.partial(
    pl.kernel,
    out_type=(jax.ShapeDtypeStruct((_GROWS, _GK), jnp.float32),
              jax.ShapeDtypeStruct((_GROWS, _GK), jnp.float32)),
    mesh=plsc.VectorSubcoreMesh(core_axis_name="c", subcore_axis_name="s"),
    compiler_params=pltpu.CompilerParams(needs_layout_passes=False),
    scratch_types=[
        pltpu.VMEM((2, _SEC), jnp.int32),
        pltpu.VMEM((_B + 2 * _L,), jnp.int32),
        pltpu.VMEM((_B + 2 * _L,), jnp.int32),
        pltpu.VMEM((_RING, _K, _PW), jnp.float32),
        pltpu.VMEM((2, _FLUSH, _GK), jnp.float32),
        pltpu.VMEM((_FLUSH + _L,), jnp.int32),
        pltpu.VMEM((2, _FLUSH), jnp.int32),
        pltpu.VMEM((_L,), jnp.int32),
        pltpu.VMEM((_L,), jnp.int32),
        pltpu.SemaphoreType.DMA((_RING,)),
        pltpu.SemaphoreType.DMA((2,)),
        pltpu.SemaphoreType.DMA((2,)),
    ],
)(_gather_body)


def _loss_body(user_h, item_h, rating_h, qg_h, pg_h, bu_h, bi_h, out_h,
               uidx, iidx, qrows, prows, bu, bi, rat, stage, sems):
    wid = lax.axis_index("s") * _NC + lax.axis_index("c")
    base = pl.multiple_of(wid * _BPW, _BPW)
    lane = lax.iota(jnp.int32, _L)

    def fire_rows(g, slot):
        rb = pl.multiple_of(base + g * 128, 128)
        return [
            pltpu.async_copy(qg_h.at[pl.ds(rb, 128)], qrows.at[slot],
                             sems.at[0]),
            pltpu.async_copy(pg_h.at[pl.ds(rb, 128)], prows.at[slot],
                             sems.at[1]),
        ]

    cps = []
    row_cps = {0: fire_rows(0, 0)}
    for g in range(4):
        pltpu.sync_copy(user_h.at[pl.ds(base + g * 128, 128)], uidx.at[g])
        pltpu.sync_copy(item_h.at[pl.ds(base + g * 128, 128)], iidx.at[g])
    pltpu.sync_copy(rating_h.at[pl.ds(base, _BPW)], rat)

    for ref, n in ((uidx, _N), (iidx, _N)):
        for g in range(4):
            for c in range(128 // _L):
                v = ref[g, pl.ds(c * _L, _L)] - 1
                ref[g, pl.ds(c * _L, _L)] = jnp.where(v < 0, n - 1, v)

    for g in range(4):
        dst = pl.ds(g * 128, 128)
        cps.append(pltpu.async_copy(bu_h.at[uidx.at[g]], bu.at[dst],
                                    sems.at[2]))
        cps.append(pltpu.async_copy(bi_h.at[iidx.at[g]], bi.at[dst],
                                    sems.at[2]))
    for cp in cps:
        cp.wait()

    sse = jnp.zeros((_L,), jnp.float32)
    for g in range(4):
        slot = g % 2
        if g + 1 < 4:
            row_cps[g + 1] = fire_rows(g + 1, (g + 1) % 2)
        for cp in row_cps.pop(g):
            cp.wait()

        def block_body(b, sse, g=g, slot=slot):
            rb = pl.multiple_of(b * _L, _L)
            dv = jnp.zeros((_L,), jnp.float32)
            for l in range(_L):
                r = rb + l
                acc = (qrows[slot, r, pl.ds(0, _L)]
                       * prows[slot, r, pl.ds(0, _L)])
                for c in range(1, _K // _L):
                    acc = acc + (qrows[slot, r, pl.ds(c * _L, _L)]
                                 * prows[slot, r, pl.ds(c * _L, _L)])
                dv = jnp.where(lane == l, jnp.sum(acc), dv)
            gb = pl.multiple_of(g * 128 + rb, _L)
            ev = (dv + bu[pl.ds(gb, _L)] + bi[pl.ds(gb, _L)]
                  + _RATING_MEAN - rat[pl.ds(gb, _L)])
            return sse + ev * ev

        sse = lax.fori_loop(0, 128 // _L, block_body, sse)
    stage[...] = sse
    pltpu.sync_copy(stage, out_h.at[pl.ds(wid * _L, _L)])


_mf_loss = functools.partial(
    pl.kernel,
    out_type=jax.ShapeDtypeStruct((_NW * _L,), jnp.float32),
    mesh=plsc.VectorSubcoreMesh(core_axis_name="c", subcore_axis_name="s"),
    compiler_params=pltpu.CompilerParams(needs_layout_passes=False),
    scratch_types=[
        pltpu.VMEM((4, 128), jnp.int32),
        pltpu.VMEM((4, 128), jnp.int32),
        pltpu.VMEM((2, 128, _GK), jnp.float32),
        pltpu.VMEM((2, 128, _GK), jnp.float32),
        pltpu.VMEM((_BPW,), jnp.float32),
        pltpu.VMEM((_BPW,), jnp.float32),
        pltpu.VMEM((_BPW,), jnp.float32),
        pltpu.VMEM((_L,), jnp.float32),
        pltpu.SemaphoreType.DMA((3,)),
    ],
)(_loss_body)


@jax.jit
def kernel(user, item, rating, Q, P, bias_users, bias_items):
    qg, pg = _mf_gather(user, item, Q.T, P.T)
    partials = _mf_loss(user, item, rating, qg, pg, bias_users, bias_items)
    return jnp.sum(partials) / _B
